# Initial kernel scaffold; baseline (speedup 1.0000x reference)
import jax
import jax.numpy as jnp
from jax.experimental import pallas as pl


def kernel(vector, table, pe):
    raise NotImplementedError("write your pallas kernel here")


# speedup vs baseline: 1.1708x; 1.1708x over previous
import jax
import jax.numpy as jnp
from jax import lax
from jax.experimental import pallas as pl
from jax.experimental.pallas import tpu as pltpu
from jax.experimental.pallas import tpu_sc as plsc

_D = 32
_W = 200
_NC = 2
_NS = 16
_NW = _NC * _NS
_CHUNK = 800
_G = 100
_GPC = _CHUNK // _G


def _embed_body(table_hbm, idx_hbm, pe_hbm, out_hbm, idx_v, pe_v, rows_v, sem):
    n_total = idx_hbm.shape[0] * idx_hbm.shape[1]
    per_w = n_total // _NW
    n_chunks = per_w // _CHUNK
    wid = lax.axis_index("s") * _NC + lax.axis_index("c")

    pltpu.sync_copy(pe_hbm, pe_v)

    base_row = wid * (per_w // _G)

    @pl.loop(0, n_chunks)
    def _chunk(j):
        row_off = base_row + j * _GPC
        pltpu.sync_copy(idx_hbm.at[pl.ds(row_off, _GPC)], idx_v)
        copies = [
            pltpu.async_copy(
                table_hbm.at[idx_v.at[g]],
                rows_v.at[pl.ds(g * _G, _G)],
                sem,
            )
            for g in range(_GPC)
        ]
        for c in copies:
            c.wait()

        @pl.loop(0, _CHUNK, unroll=8)
        def _row(r):
            rows_v[r, 0:16] = rows_v[r, 0:16] + pe_v[r, 0:16]
            rows_v[r, 16:32] = rows_v[r, 16:32] + pe_v[r, 16:32]

        out_off = wid * per_w + j * _CHUNK
        pltpu.sync_copy(rows_v, out_hbm.at[pl.ds(out_off, _CHUNK)])


def _make_sc_call(n_total):
    mesh = plsc.VectorSubcoreMesh(
        core_axis_name="c", subcore_axis_name="s",
        num_cores=_NC, num_subcores=_NS,
    )
    return pl.kernel(
        _embed_body,
        out_type=jax.ShapeDtypeStruct((n_total, _D), jnp.float32),
        mesh=mesh,
        scratch_types=[
            pltpu.VMEM((_GPC, _G), jnp.int32),
            pltpu.VMEM((_CHUNK, _D), jnp.float32),
            pltpu.VMEM((_CHUNK, _D), jnp.float32),
            pltpu.SemaphoreType.DMA,
        ],
        compiler_params=pltpu.CompilerParams(use_tc_tiling_on_sc=False),
    )


def kernel(vector, table, pe):
    b, w = vector.shape
    n_total = b * w
    idx = vector.reshape(n_total // _G, _G).astype(jnp.int32)
    pe_tile = jnp.tile(pe, (_CHUNK // _W, 1))
    out = _make_sc_call(n_total)(table, idx, pe_tile)
    return out.reshape(b, w, _D)


# speedup vs baseline: 1.2421x; 1.0609x over previous
import jax
import jax.numpy as jnp
from jax import lax
from jax.experimental import pallas as pl
from jax.experimental.pallas import tpu as pltpu
from jax.experimental.pallas import tpu_sc as plsc

_D = 32
_W = 200
_NC = 2
_NS = 16
_NW = _NC * _NS
_CHUNK = 800
_G = 100
_GPC = _CHUNK // _G


def _embed_body(table_hbm, idx_hbm, pe_hbm, out_hbm,
                idx_v, pe_v, rows0, rows1, sem0, sem1):
    n_total = idx_hbm.shape[0] * idx_hbm.shape[1]
    per_w = n_total // _NW
    n_chunks = per_w // _CHUNK
    rows_per_tile = per_w // _G
    wid = lax.axis_index("s") * _NC + lax.axis_index("c")

    pltpu.sync_copy(idx_hbm.at[pl.ds(wid * rows_per_tile, rows_per_tile)], idx_v)
    pltpu.sync_copy(pe_hbm, pe_v)

    def fire(j, rows_v, sem):
        for g in range(_GPC):
            pltpu.async_copy(
                table_hbm.at[idx_v.at[j * _GPC + g]],
                rows_v.at[pl.ds(g * _G, _G)],
                sem,
            )

    def drain(rows_v, sem):
        pltpu.make_async_copy(table_hbm.at[pl.ds(0, _CHUNK)], rows_v, sem).wait()

    def finish(j, rows_v, sem):
        drain(rows_v, sem)

        @pl.loop(0, _CHUNK, unroll=8)
        def _row(r):
            rows_v[r, 0:16] = rows_v[r, 0:16] + pe_v[r, 0:16]
            rows_v[r, 16:32] = rows_v[r, 16:32] + pe_v[r, 16:32]

        pltpu.sync_copy(rows_v, out_hbm.at[pl.ds(wid * per_w + j * _CHUNK, _CHUNK)])

    fire(0, rows0, sem0)

    @pl.loop(0, n_chunks // 2)
    def _pair(i):
        fire(2 * i + 1, rows1, sem1)
        finish(2 * i, rows0, sem0)

        @pl.when(i < n_chunks // 2 - 1)
        def _():
            fire(2 * i + 2, rows0, sem0)

        finish(2 * i + 1, rows1, sem1)


def _make_sc_call(n_total):
    per_w = n_total // _NW
    mesh = plsc.VectorSubcoreMesh(
        core_axis_name="c", subcore_axis_name="s",
        num_cores=_NC, num_subcores=_NS,
    )
    return pl.kernel(
        _embed_body,
        out_type=jax.ShapeDtypeStruct((n_total, _D), jnp.float32),
        mesh=mesh,
        scratch_types=[
            pltpu.VMEM((per_w // _G, _G), jnp.int32),
            pltpu.VMEM((_CHUNK, _D), jnp.float32),
            pltpu.VMEM((_CHUNK, _D), jnp.float32),
            pltpu.VMEM((_CHUNK, _D), jnp.float32),
            pltpu.SemaphoreType.DMA,
            pltpu.SemaphoreType.DMA,
        ],
        compiler_params=pltpu.CompilerParams(use_tc_tiling_on_sc=False),
    )


def kernel(vector, table, pe):
    b, w = vector.shape
    n_total = b * w
    idx = vector.reshape(n_total // _G, _G).astype(jnp.int32)
    pe_tile = jnp.tile(pe, (_CHUNK // _W, 1))
    out = _make_sc_call(n_total)(table, idx, pe_tile)
    return out.reshape(b, w, _D)
